# baseline (speedup 1.0000x reference)
import functools

import jax
import jax.numpy as jnp
from jax import lax
from jax.experimental import pallas as pl
from jax.experimental.pallas import tpu as pltpu
from jax.experimental.pallas import tpu_sc as plsc

NUM_SELECT = 1000
_NB = 4000
_NBP = 4096


def _softmax_thresh_body(logits_ref, probs_ref, xt_ref):
    C = logits_ref.shape[2]
    x = logits_ref[0]
    xt_ref[...] = jnp.swapaxes(x, 0, 1)

    m = xt_ref[0]
    for j in range(1, C):
        m = jnp.maximum(m, xt_ref[j])
    s = None
    for st in range(0, C, 13):
        g = jnp.exp(xt_ref[st] - m)
        for j in range(st + 1, min(st + 13, C)):
            g = g + jnp.exp(xt_ref[j] - m)
        s = g if s is None else s + g
    prob = jnp.exp(xt_ref[C - 1] - m) / s
    padded = jnp.pad(prob, (0, _NBP - _NB))
    probs_ref[...] = padded.reshape(1, 1, 1, _NBP)


def _softmax_thresh(logits):
    B, N, C = logits.shape
    nchunks = N // _NB
    return pl.pallas_call(
        _softmax_thresh_body,
        grid=(B, nchunks),
        in_specs=[pl.BlockSpec((1, _NB, C), lambda b, n: (b, n, 0))],
        out_specs=[
            pl.BlockSpec((1, 1, 1, _NBP), lambda b, n: (b, n, 0, 0)),
        ],
        out_shape=[
            jax.ShapeDtypeStruct((B, nchunks, 1, _NBP), jnp.float32),
        ],
        scratch_shapes=[
            pltpu.VMEM((C, _NB), jnp.float32),
        ],
        compiler_params=pltpu.CompilerParams(
            dimension_semantics=("parallel", "parallel")),
    )(logits)[0]


def _thresh_body(probs_ref, tbits_ref):
    u = pltpu.bitcast(probs_ref[...], jnp.int32)

    def step(i, t):
        cand = t | (1 << (30 - i))
        cnt = jnp.sum((u >= cand).astype(jnp.int32), axis=1, keepdims=True)
        return jnp.where(cnt >= NUM_SELECT, cand, t)

    t = jax.lax.fori_loop(0, 31, step, jnp.zeros((u.shape[0], 1), jnp.int32))
    tbits_ref[...] = jnp.broadcast_to(t[:, :, None], tbits_ref.shape)


def _thresh(probs_pad):
    B, NPAD = probs_pad.shape
    return pl.pallas_call(
        _thresh_body,
        out_shape=jax.ShapeDtypeStruct((B, 1, 128), jnp.int32),
    )(probs_pad)


_K = NUM_SELECT
_CAP = 1024
_BUF = 1056


def _sc_body(probs_hbm, tbits_hbm, boxes_hbm, scale_hbm, cls_hbm, lbl_hbm,
             out_hbm,
             prob_v, tb_v, scl_v, cls_v, lbl_v, box_v,
             ka, ia, kc, ic, hist, base, gidx, score, outv, tmp16):
    wid = lax.axis_index("s") * 2 + lax.axis_index("c")

    @pl.when(wid < 16)
    def _():
        t = wid
        pltpu.sync_copy(probs_hbm.at[t], prob_v)
        pltpu.sync_copy(boxes_hbm.at[t], box_v)
        pltpu.sync_copy(tbits_hbm.at[t], tb_v)
        pltpu.sync_copy(cls_hbm.at[t], cls_v)
        pltpu.sync_copy(lbl_hbm.at[t], lbl_v)
        pltpu.sync_copy(scale_hbm, scl_v)

        lanes = lax.iota(jnp.int32, 16)
        tvec = tb_v[pl.ds(0, 16)]

        def compact(i, off):
            u = plsc.bitcast(prob_v[pl.ds(i * 16, 16)], jnp.int32)
            m = u >= tvec
            ks = ~u
            idx = i * 16 + lanes

            @pl.when(off < _CAP - 16)
            def _():
                plsc.store_compressed(ka.at[pl.ds(off, 16)], ks, mask=m)
                plsc.store_compressed(ia.at[pl.ds(off, 16)], idx, mask=m)

            return off + jnp.max(plsc.all_reduce_population_count(m))

        off = lax.fori_loop(0, prob_v.shape[0] // 16, compact, jnp.int32(0))

        def sentinel(j, _):
            @pl.when(off + j * 16 < _CAP + 16)
            def _():
                ka[pl.ds(off + j * 16, 16)] = jnp.full((16,), -1, jnp.int32)
                ia[pl.ds(off + j * 16, 16)] = jnp.zeros((16,), jnp.int32)
            return 0

        lax.fori_loop(0, 4, sentinel, 0)

        for p in range(6):
            src_k, src_i = (ka, ia) if p % 2 == 0 else (kc, ic)
            dst_k, dst_i = (kc, ic) if p % 2 == 0 else (ka, ia)
            sh = 5 * p
            hist[pl.ds(0, 16)] = jnp.zeros((16,), jnp.int32)
            hist[pl.ds(16, 16)] = jnp.zeros((16,), jnp.int32)

            def histo(v, _, src_k=src_k, sh=sh):
                k = plsc.bitcast(src_k[pl.ds(v * 16, 16)], jnp.uint32)
                d = plsc.bitcast((k >> sh) & 31, jnp.int32)
                cnt, lastm = plsc.scan_count(d)
                plsc.addupdate_scatter(hist, [d], cnt, mask=lastm)
                return 0

            lax.fori_loop(0, _CAP // 16, histo, 0)
            h0 = hist[pl.ds(0, 16)]
            h1 = hist[pl.ds(16, 16)]
            e0 = plsc.cumsum(h0) - h0
            tot0 = jnp.max(plsc.cumsum(h0))
            e1 = plsc.cumsum(h1) - h1 + tot0
            base[pl.ds(0, 16)] = e0
            base[pl.ds(16, 16)] = e1

            def permute(v, _, src_k=src_k, src_i=src_i, dst_k=dst_k,
                        dst_i=dst_i, sh=sh):
                k = src_k[pl.ds(v * 16, 16)]
                i_ = src_i[pl.ds(v * 16, 16)]
                d = plsc.bitcast((plsc.bitcast(k, jnp.uint32) >> sh) & 31,
                                 jnp.int32)
                cnt, lastm = plsc.scan_count(d)
                pos = plsc.load_gather(base, [d]) + (cnt - 1)
                plsc.store_scatter(dst_k, [pos], k)
                plsc.store_scatter(dst_i, [pos], i_)
                plsc.addupdate_scatter(base, [d], cnt, mask=lastm)
                return 0

            lax.fori_loop(0, _CAP // 16, permute, 0)
        res_k, res_i = ka, ia

        def post(v, _):
            ks = res_k[pl.ds(v * 16, 16)]
            prob = plsc.bitcast(~ks, jnp.float32)
            score[pl.ds(v * 16, 16)] = prob
            pidx = res_i[pl.ds(v * 16, 16)]
            orig = pidx - ((pidx >> 12) * 96)
            gidx[pl.ds(v * 16, 16)] = orig
            return 0

        lax.fori_loop(0, _CAP // 16, post, 0)

        cid = jnp.zeros((16,), jnp.int32)
        for j in range(3, -1, -1):
            cls = cls_v[pl.ds(j * 16, 16)]
            lbl = lbl_v[pl.ds(j * 16, 16)]
            ids = jnp.where(lbl == 1, cls, jnp.zeros((16,), jnp.int32))
            m = ids != 0
            ffs = plsc.all_reduce_ffs(m)
            tmp16[pl.ds(0, 16)] = ids
            val = plsc.load_gather(tmp16, [jnp.minimum(ffs, 15)])
            cid = jnp.where(ffs < 16, val, cid)
        cid_f = cid.astype(jnp.float32)

        sh_s = plsc.load_gather(scl_v, [jnp.zeros((16,), jnp.int32),
                                        jnp.full((16,), t, jnp.int32)])
        sw_s = plsc.load_gather(scl_v, [jnp.ones((16,), jnp.int32),
                                        jnp.full((16,), t, jnp.int32)])

        def assemble(v, _):
            flat = v * 16 + lanes
            q = flat // 6
            c = flat - q * 6
            sc_g = plsc.load_gather(score, [q])
            ca = (c - 2) & 1
            cb = ca + 2
            oq = plsc.load_gather(gidx, [q])
            g1 = plsc.load_gather(box_v, [oq * 4 + ca])
            g2 = plsc.load_gather(box_v, [oq * 4 + cb])
            sgn = jnp.where(c < 4, jnp.full((16,), -0.5, jnp.float32),
                            jnp.full((16,), 0.5, jnp.float32))
            scl_l = jnp.where(ca == 0, sw_s, sh_s)
            bx = (g1 + sgn * g2) * scl_l
            val = jnp.where(c == 0, sc_g, jnp.where(c == 1, cid_f, bx))
            outv[pl.ds(v * 16, 16)] = val
            return 0

        lax.fori_loop(0, (_K * 6) // 16, assemble, 0)
        pltpu.sync_copy(outv.at[pl.ds(0, _K * 6)],
                        out_hbm.at[pl.ds(t * (_K * 6), _K * 6)])


def _topk_sc(probs_pad, tbits, boxes_flat, scale_cols, classes_pad, labels_pad):
    B = probs_pad.shape[0]
    mesh = plsc.VectorSubcoreMesh(core_axis_name="c", subcore_axis_name="s")
    return pl.kernel(
        _sc_body,
        out_type=jax.ShapeDtypeStruct((B * _K * 6,), jnp.float32),
        mesh=mesh,
        compiler_params=pltpu.CompilerParams(needs_layout_passes=False),
        scratch_types=[
            pltpu.VMEM((probs_pad.shape[1],), jnp.float32),
            pltpu.VMEM((128,), jnp.int32),
            pltpu.VMEM((2, 16), jnp.float32),
            pltpu.VMEM((64,), jnp.int32),
            pltpu.VMEM((64,), jnp.int32),
            pltpu.VMEM((80000,), jnp.float32),
            pltpu.VMEM((_BUF,), jnp.int32),
            pltpu.VMEM((_BUF,), jnp.int32),
            pltpu.VMEM((_CAP,), jnp.int32),
            pltpu.VMEM((_CAP,), jnp.int32),
            pltpu.VMEM((32,), jnp.int32),
            pltpu.VMEM((32,), jnp.int32),
            pltpu.VMEM((_CAP,), jnp.int32),
            pltpu.VMEM((_CAP,), jnp.float32),
            pltpu.VMEM((_K * 6 + 96,), jnp.float32),
            pltpu.VMEM((16,), jnp.int32),
        ],
    )(probs_pad, tbits, boxes_flat, scale_cols, classes_pad, labels_pad)


def kernel(pred_class_logits, pred_sim_logits, pred_boxes, orig_size, classes, labels):
    del pred_sim_logits
    B, N, C = pred_class_logits.shape
    probs_pad = _softmax_thresh(pred_class_logits)
    probs_pad = probs_pad.reshape(B, -1)
    tbits = _thresh(probs_pad).reshape(B, 128)

    boxes_flat = pred_boxes.reshape(B, N * 4)
    scale_cols = jnp.stack([orig_size[:, 0], orig_size[:, 1]], axis=0).astype(jnp.float32)
    classes_pad = jnp.pad(classes, ((0, 0), (0, 64 - classes.shape[1])))
    labels_pad = jnp.pad(labels, ((0, 0), (0, 64 - labels.shape[1])))

    out = _topk_sc(probs_pad, tbits, boxes_flat, scale_cols, classes_pad, labels_pad)
    return out.reshape(B, _K, 6)


# speedup vs baseline: 1.0799x; 1.0799x over previous
import functools

import jax
import jax.numpy as jnp
from jax import lax
from jax.experimental import pallas as pl
from jax.experimental.pallas import tpu as pltpu
from jax.experimental.pallas import tpu_sc as plsc

NUM_SELECT = 1000
_NB = 10000
_NBP = 10240


def _softmax_thresh_body(logits_ref, probs_ref, xt_ref):
    C = logits_ref.shape[2]
    x = logits_ref[0]
    xt_ref[...] = jnp.swapaxes(x, 0, 1)

    m = xt_ref[0]
    for j in range(1, C):
        m = jnp.maximum(m, xt_ref[j])
    s = None
    for st in range(0, C, 13):
        g = jnp.exp(xt_ref[st] - m)
        for j in range(st + 1, min(st + 13, C)):
            g = g + jnp.exp(xt_ref[j] - m)
        s = g if s is None else s + g
    prob = jnp.exp(xt_ref[C - 1] - m) / s
    padded = jnp.pad(prob, (0, _NBP - _NB))
    probs_ref[...] = padded.reshape(1, 1, 1, _NBP)


def _softmax_thresh(logits):
    B, N, C = logits.shape
    nchunks = N // _NB
    return pl.pallas_call(
        _softmax_thresh_body,
        grid=(B, nchunks),
        in_specs=[pl.BlockSpec((1, _NB, C), lambda b, n: (b, n, 0))],
        out_specs=[
            pl.BlockSpec((1, 1, 1, _NBP), lambda b, n: (b, n, 0, 0)),
        ],
        out_shape=[
            jax.ShapeDtypeStruct((B, nchunks, 1, _NBP), jnp.float32),
        ],
        scratch_shapes=[
            pltpu.VMEM((C, _NB), jnp.float32),
        ],
        compiler_params=pltpu.CompilerParams(
            dimension_semantics=("parallel", "parallel")),
    )(logits)[0]


def _thresh_body(probs_ref, tbits_ref):
    u = pltpu.bitcast(probs_ref[...], jnp.int32)

    def step(i, t):
        cand = t | (1 << (30 - i))
        cnt = jnp.sum((u >= cand).astype(jnp.int32), axis=1, keepdims=True)
        return jnp.where(cnt >= NUM_SELECT, cand, t)

    t = jax.lax.fori_loop(0, 31, step, jnp.zeros((u.shape[0], 1), jnp.int32))
    tbits_ref[...] = jnp.broadcast_to(t[:, :, None], tbits_ref.shape)


def _thresh(probs_pad):
    B, NPAD = probs_pad.shape
    return pl.pallas_call(
        _thresh_body,
        out_shape=jax.ShapeDtypeStruct((B, 1, 128), jnp.int32),
    )(probs_pad)


_K = NUM_SELECT
_CAP = 1024
_BUF = 1056


def _sc_body(probs_hbm, tbits_hbm, boxes_hbm, scale_hbm, cls_hbm, lbl_hbm,
             out_hbm,
             prob_v, tb_v, scl_v, cls_v, lbl_v, box_v,
             ka, ia, kc, ic, hist, base, gidx, score, outv, tmp16):
    wid = lax.axis_index("s") * 2 + lax.axis_index("c")

    @pl.when(wid < 16)
    def _():
        t = wid
        pltpu.sync_copy(probs_hbm.at[t], prob_v)
        pltpu.sync_copy(boxes_hbm.at[t], box_v)
        pltpu.sync_copy(tbits_hbm.at[t], tb_v)
        pltpu.sync_copy(cls_hbm.at[t], cls_v)
        pltpu.sync_copy(lbl_hbm.at[t], lbl_v)
        pltpu.sync_copy(scale_hbm, scl_v)

        lanes = lax.iota(jnp.int32, 16)
        tvec = tb_v[pl.ds(0, 16)]

        def compact(i, off):
            u = plsc.bitcast(prob_v[pl.ds(i * 16, 16)], jnp.int32)
            m = u >= tvec
            ks = ~u
            idx = i * 16 + lanes

            @pl.when(off < _CAP - 16)
            def _():
                plsc.store_compressed(ka.at[pl.ds(off, 16)], ks, mask=m)
                plsc.store_compressed(ia.at[pl.ds(off, 16)], idx, mask=m)

            return off + jnp.max(plsc.all_reduce_population_count(m))

        off = lax.fori_loop(0, prob_v.shape[0] // 16, compact, jnp.int32(0))

        def sentinel(j, _):
            @pl.when(off + j * 16 < _CAP + 16)
            def _():
                ka[pl.ds(off + j * 16, 16)] = jnp.full((16,), -1, jnp.int32)
                ia[pl.ds(off + j * 16, 16)] = jnp.zeros((16,), jnp.int32)
            return 0

        lax.fori_loop(0, 4, sentinel, 0)

        for p in range(6):
            src_k, src_i = (ka, ia) if p % 2 == 0 else (kc, ic)
            dst_k, dst_i = (kc, ic) if p % 2 == 0 else (ka, ia)
            sh = 5 * p
            hist[pl.ds(0, 16)] = jnp.zeros((16,), jnp.int32)
            hist[pl.ds(16, 16)] = jnp.zeros((16,), jnp.int32)

            def histo(v, _, src_k=src_k, sh=sh):
                k = plsc.bitcast(src_k[pl.ds(v * 16, 16)], jnp.uint32)
                d = plsc.bitcast((k >> sh) & 31, jnp.int32)
                cnt, lastm = plsc.scan_count(d)
                plsc.addupdate_scatter(hist, [d], cnt, mask=lastm)
                return 0

            lax.fori_loop(0, _CAP // 16, histo, 0)
            h0 = hist[pl.ds(0, 16)]
            h1 = hist[pl.ds(16, 16)]
            e0 = plsc.cumsum(h0) - h0
            tot0 = jnp.max(plsc.cumsum(h0))
            e1 = plsc.cumsum(h1) - h1 + tot0
            base[pl.ds(0, 16)] = e0
            base[pl.ds(16, 16)] = e1

            def permute(v, _, src_k=src_k, src_i=src_i, dst_k=dst_k,
                        dst_i=dst_i, sh=sh):
                k = src_k[pl.ds(v * 16, 16)]
                i_ = src_i[pl.ds(v * 16, 16)]
                d = plsc.bitcast((plsc.bitcast(k, jnp.uint32) >> sh) & 31,
                                 jnp.int32)
                cnt, lastm = plsc.scan_count(d)
                pos = plsc.load_gather(base, [d]) + (cnt - 1)
                plsc.store_scatter(dst_k, [pos], k)
                plsc.store_scatter(dst_i, [pos], i_)
                plsc.addupdate_scatter(base, [d], cnt, mask=lastm)
                return 0

            lax.fori_loop(0, _CAP // 16, permute, 0)
        res_k, res_i = ka, ia

        def post(v, _):
            ks = res_k[pl.ds(v * 16, 16)]
            prob = plsc.bitcast(~ks, jnp.float32)
            score[pl.ds(v * 16, 16)] = prob
            pidx = res_i[pl.ds(v * 16, 16)]
            orig = pidx - jnp.where(pidx >= _NBP,
                                    jnp.full((16,), _NBP - _NB, jnp.int32),
                                    jnp.zeros((16,), jnp.int32))
            gidx[pl.ds(v * 16, 16)] = orig
            return 0

        lax.fori_loop(0, _CAP // 16, post, 0)

        cid = jnp.zeros((16,), jnp.int32)
        for j in range(3, -1, -1):
            cls = cls_v[pl.ds(j * 16, 16)]
            lbl = lbl_v[pl.ds(j * 16, 16)]
            ids = jnp.where(lbl == 1, cls, jnp.zeros((16,), jnp.int32))
            m = ids != 0
            ffs = plsc.all_reduce_ffs(m)
            tmp16[pl.ds(0, 16)] = ids
            val = plsc.load_gather(tmp16, [jnp.minimum(ffs, 15)])
            cid = jnp.where(ffs < 16, val, cid)
        cid_f = cid.astype(jnp.float32)

        sh_s = plsc.load_gather(scl_v, [jnp.zeros((16,), jnp.int32),
                                        jnp.full((16,), t, jnp.int32)])
        sw_s = plsc.load_gather(scl_v, [jnp.ones((16,), jnp.int32),
                                        jnp.full((16,), t, jnp.int32)])

        def assemble(v, _):
            flat = v * 16 + lanes
            q = flat // 6
            c = flat - q * 6
            sc_g = plsc.load_gather(score, [q])
            ca = (c - 2) & 1
            cb = ca + 2
            oq = plsc.load_gather(gidx, [q])
            g1 = plsc.load_gather(box_v, [oq * 4 + ca])
            g2 = plsc.load_gather(box_v, [oq * 4 + cb])
            sgn = jnp.where(c < 4, jnp.full((16,), -0.5, jnp.float32),
                            jnp.full((16,), 0.5, jnp.float32))
            scl_l = jnp.where(ca == 0, sw_s, sh_s)
            bx = (g1 + sgn * g2) * scl_l
            val = jnp.where(c == 0, sc_g, jnp.where(c == 1, cid_f, bx))
            outv[pl.ds(v * 16, 16)] = val
            return 0

        lax.fori_loop(0, (_K * 6) // 16, assemble, 0)
        pltpu.sync_copy(outv.at[pl.ds(0, _K * 6)],
                        out_hbm.at[pl.ds(t * (_K * 6), _K * 6)])


def _topk_sc(probs_pad, tbits, boxes_flat, scale_cols, classes_pad, labels_pad):
    B = probs_pad.shape[0]
    mesh = plsc.VectorSubcoreMesh(core_axis_name="c", subcore_axis_name="s")
    return pl.kernel(
        _sc_body,
        out_type=jax.ShapeDtypeStruct((B * _K * 6,), jnp.float32),
        mesh=mesh,
        compiler_params=pltpu.CompilerParams(needs_layout_passes=False),
        scratch_types=[
            pltpu.VMEM((probs_pad.shape[1],), jnp.float32),
            pltpu.VMEM((128,), jnp.int32),
            pltpu.VMEM((2, 16), jnp.float32),
            pltpu.VMEM((64,), jnp.int32),
            pltpu.VMEM((64,), jnp.int32),
            pltpu.VMEM((80000,), jnp.float32),
            pltpu.VMEM((_BUF,), jnp.int32),
            pltpu.VMEM((_BUF,), jnp.int32),
            pltpu.VMEM((_CAP,), jnp.int32),
            pltpu.VMEM((_CAP,), jnp.int32),
            pltpu.VMEM((32,), jnp.int32),
            pltpu.VMEM((32,), jnp.int32),
            pltpu.VMEM((_CAP,), jnp.int32),
            pltpu.VMEM((_CAP,), jnp.float32),
            pltpu.VMEM((_K * 6 + 96,), jnp.float32),
            pltpu.VMEM((16,), jnp.int32),
        ],
    )(probs_pad, tbits, boxes_flat, scale_cols, classes_pad, labels_pad)


def kernel(pred_class_logits, pred_sim_logits, pred_boxes, orig_size, classes, labels):
    del pred_sim_logits
    B, N, C = pred_class_logits.shape
    probs_pad = _softmax_thresh(pred_class_logits)
    probs_pad = probs_pad.reshape(B, -1)
    tbits = _thresh(probs_pad).reshape(B, 128)

    boxes_flat = pred_boxes.reshape(B, N * 4)
    scale_cols = jnp.stack([orig_size[:, 0], orig_size[:, 1]], axis=0).astype(jnp.float32)
    classes_pad = jnp.pad(classes, ((0, 0), (0, 64 - classes.shape[1])))
    labels_pad = jnp.pad(labels, ((0, 0), (0, 64 - labels.shape[1])))

    out = _topk_sc(probs_pad, tbits, boxes_flat, scale_cols, classes_pad, labels_pad)
    return out.reshape(B, _K, 6)


# speedup vs baseline: 1.1427x; 1.0581x over previous
import functools

import jax
import jax.numpy as jnp
from jax import lax
from jax.experimental import pallas as pl
from jax.experimental.pallas import tpu as pltpu
from jax.experimental.pallas import tpu_sc as plsc

NUM_SELECT = 1000
_NB = 20000
_NBP = 20480


def _softmax_thresh_body(logits_ref, probs_ref, xt_ref):
    C = logits_ref.shape[2]
    x = logits_ref[0]
    xt_ref[...] = jnp.swapaxes(x, 0, 1)

    m = xt_ref[0]
    for j in range(1, C):
        m = jnp.maximum(m, xt_ref[j])
    s = None
    for st in range(0, C, 13):
        g = jnp.exp(xt_ref[st] - m)
        for j in range(st + 1, min(st + 13, C)):
            g = g + jnp.exp(xt_ref[j] - m)
        s = g if s is None else s + g
    prob = jnp.exp(xt_ref[C - 1] - m) / s
    padded = jnp.pad(prob, (0, _NBP - _NB))
    probs_ref[...] = padded.reshape(1, 1, 1, _NBP)


def _softmax_thresh(logits):
    B, N, C = logits.shape
    nchunks = N // _NB
    return pl.pallas_call(
        _softmax_thresh_body,
        grid=(B, nchunks),
        in_specs=[pl.BlockSpec((1, _NB, C), lambda b, n: (b, n, 0))],
        out_specs=[
            pl.BlockSpec((1, 1, 1, _NBP), lambda b, n: (b, n, 0, 0)),
        ],
        out_shape=[
            jax.ShapeDtypeStruct((B, nchunks, 1, _NBP), jnp.float32),
        ],
        scratch_shapes=[
            pltpu.VMEM((C, _NB), jnp.float32),
        ],
        compiler_params=pltpu.CompilerParams(
            dimension_semantics=("parallel", "parallel")),
    )(logits)[0]


def _thresh_body(probs_ref, tbits_ref):
    u = pltpu.bitcast(probs_ref[...], jnp.int32)

    def step(i, t):
        cand = t | (1 << (30 - i))
        cnt = jnp.sum((u >= cand).astype(jnp.int32), axis=1, keepdims=True)
        return jnp.where(cnt >= NUM_SELECT, cand, t)

    t = jax.lax.fori_loop(0, 31, step, jnp.zeros((u.shape[0], 1), jnp.int32))
    tbits_ref[...] = jnp.broadcast_to(t[:, :, None], tbits_ref.shape)


def _thresh(probs_pad):
    B, NPAD = probs_pad.shape
    return pl.pallas_call(
        _thresh_body,
        out_shape=jax.ShapeDtypeStruct((B, 1, 128), jnp.int32),
    )(probs_pad)


_K = NUM_SELECT
_CAP = 1024
_BUF = 1056


def _sc_body(probs_hbm, tbits_hbm, boxes_hbm, scale_hbm, cls_hbm, lbl_hbm,
             out_hbm,
             prob_v, tb_v, scl_v, cls_v, lbl_v, box_v,
             ka, ia, kc, ic, hist, base, gidx, score, outv, tmp16):
    wid = lax.axis_index("s") * 2 + lax.axis_index("c")

    @pl.when(wid < 16)
    def _():
        t = wid
        pltpu.sync_copy(probs_hbm.at[t], prob_v)
        pltpu.sync_copy(boxes_hbm.at[t], box_v)
        pltpu.sync_copy(tbits_hbm.at[t], tb_v)
        pltpu.sync_copy(cls_hbm.at[t], cls_v)
        pltpu.sync_copy(lbl_hbm.at[t], lbl_v)
        pltpu.sync_copy(scale_hbm, scl_v)

        lanes = lax.iota(jnp.int32, 16)
        tvec = tb_v[pl.ds(0, 16)]

        def compact(i, off):
            u = plsc.bitcast(prob_v[pl.ds(i * 16, 16)], jnp.int32)
            m = u >= tvec
            ks = ~u
            idx = i * 16 + lanes

            @pl.when(off < _CAP - 16)
            def _():
                plsc.store_compressed(ka.at[pl.ds(off, 16)], ks, mask=m)
                plsc.store_compressed(ia.at[pl.ds(off, 16)], idx, mask=m)

            return off + jnp.max(plsc.all_reduce_population_count(m))

        off = lax.fori_loop(0, prob_v.shape[0] // 16, compact, jnp.int32(0))

        def sentinel(j, _):
            @pl.when(off + j * 16 < _CAP + 16)
            def _():
                ka[pl.ds(off + j * 16, 16)] = jnp.full((16,), -1, jnp.int32)
                ia[pl.ds(off + j * 16, 16)] = jnp.zeros((16,), jnp.int32)
            return 0

        lax.fori_loop(0, 4, sentinel, 0)

        for p in range(6):
            src_k, src_i = (ka, ia) if p % 2 == 0 else (kc, ic)
            dst_k, dst_i = (kc, ic) if p % 2 == 0 else (ka, ia)
            sh = 5 * p
            hist[pl.ds(0, 16)] = jnp.zeros((16,), jnp.int32)
            hist[pl.ds(16, 16)] = jnp.zeros((16,), jnp.int32)

            def histo(v, _, src_k=src_k, sh=sh):
                k = plsc.bitcast(src_k[pl.ds(v * 16, 16)], jnp.uint32)
                d = plsc.bitcast((k >> sh) & 31, jnp.int32)
                cnt, lastm = plsc.scan_count(d)
                plsc.addupdate_scatter(hist, [d], cnt, mask=lastm)
                return 0

            lax.fori_loop(0, _CAP // 16, histo, 0)
            h0 = hist[pl.ds(0, 16)]
            h1 = hist[pl.ds(16, 16)]
            e0 = plsc.cumsum(h0) - h0
            tot0 = jnp.max(plsc.cumsum(h0))
            e1 = plsc.cumsum(h1) - h1 + tot0
            base[pl.ds(0, 16)] = e0
            base[pl.ds(16, 16)] = e1

            def permute(v, _, src_k=src_k, src_i=src_i, dst_k=dst_k,
                        dst_i=dst_i, sh=sh):
                k = src_k[pl.ds(v * 16, 16)]
                i_ = src_i[pl.ds(v * 16, 16)]
                d = plsc.bitcast((plsc.bitcast(k, jnp.uint32) >> sh) & 31,
                                 jnp.int32)
                cnt, lastm = plsc.scan_count(d)
                pos = plsc.load_gather(base, [d]) + (cnt - 1)
                plsc.store_scatter(dst_k, [pos], k)
                plsc.store_scatter(dst_i, [pos], i_)
                plsc.addupdate_scatter(base, [d], cnt, mask=lastm)
                return 0

            lax.fori_loop(0, _CAP // 16, permute, 0)
        res_k, res_i = ka, ia

        def post(v, _):
            ks = res_k[pl.ds(v * 16, 16)]
            prob = plsc.bitcast(~ks, jnp.float32)
            score[pl.ds(v * 16, 16)] = prob
            pidx = res_i[pl.ds(v * 16, 16)]
            orig = pidx - jnp.where(pidx >= _NBP,
                                    jnp.full((16,), _NBP - _NB, jnp.int32),
                                    jnp.zeros((16,), jnp.int32))
            gidx[pl.ds(v * 16, 16)] = orig
            return 0

        lax.fori_loop(0, _CAP // 16, post, 0)

        cid = jnp.zeros((16,), jnp.int32)
        for j in range(3, -1, -1):
            cls = cls_v[pl.ds(j * 16, 16)]
            lbl = lbl_v[pl.ds(j * 16, 16)]
            ids = jnp.where(lbl == 1, cls, jnp.zeros((16,), jnp.int32))
            m = ids != 0
            ffs = plsc.all_reduce_ffs(m)
            tmp16[pl.ds(0, 16)] = ids
            val = plsc.load_gather(tmp16, [jnp.minimum(ffs, 15)])
            cid = jnp.where(ffs < 16, val, cid)
        cid_f = cid.astype(jnp.float32)

        sh_s = plsc.load_gather(scl_v, [jnp.zeros((16,), jnp.int32),
                                        jnp.full((16,), t, jnp.int32)])
        sw_s = plsc.load_gather(scl_v, [jnp.ones((16,), jnp.int32),
                                        jnp.full((16,), t, jnp.int32)])

        def assemble(v, _):
            flat = v * 16 + lanes
            q = flat // 6
            c = flat - q * 6
            sc_g = plsc.load_gather(score, [q])
            ca = (c - 2) & 1
            cb = ca + 2
            oq = plsc.load_gather(gidx, [q])
            g1 = plsc.load_gather(box_v, [oq * 4 + ca])
            g2 = plsc.load_gather(box_v, [oq * 4 + cb])
            sgn = jnp.where(c < 4, jnp.full((16,), -0.5, jnp.float32),
                            jnp.full((16,), 0.5, jnp.float32))
            scl_l = jnp.where(ca == 0, sw_s, sh_s)
            bx = (g1 + sgn * g2) * scl_l
            val = jnp.where(c == 0, sc_g, jnp.where(c == 1, cid_f, bx))
            outv[pl.ds(v * 16, 16)] = val
            return 0

        lax.fori_loop(0, (_K * 6) // 16, assemble, 0)
        pltpu.sync_copy(outv.at[pl.ds(0, _K * 6)],
                        out_hbm.at[pl.ds(t * (_K * 6), _K * 6)])


def _topk_sc(probs_pad, tbits, boxes_flat, scale_cols, classes_pad, labels_pad):
    B = probs_pad.shape[0]
    mesh = plsc.VectorSubcoreMesh(core_axis_name="c", subcore_axis_name="s")
    return pl.kernel(
        _sc_body,
        out_type=jax.ShapeDtypeStruct((B * _K * 6,), jnp.float32),
        mesh=mesh,
        compiler_params=pltpu.CompilerParams(needs_layout_passes=False),
        scratch_types=[
            pltpu.VMEM((probs_pad.shape[1],), jnp.float32),
            pltpu.VMEM((128,), jnp.int32),
            pltpu.VMEM((2, 16), jnp.float32),
            pltpu.VMEM((64,), jnp.int32),
            pltpu.VMEM((64,), jnp.int32),
            pltpu.VMEM((80000,), jnp.float32),
            pltpu.VMEM((_BUF,), jnp.int32),
            pltpu.VMEM((_BUF,), jnp.int32),
            pltpu.VMEM((_CAP,), jnp.int32),
            pltpu.VMEM((_CAP,), jnp.int32),
            pltpu.VMEM((32,), jnp.int32),
            pltpu.VMEM((32,), jnp.int32),
            pltpu.VMEM((_CAP,), jnp.int32),
            pltpu.VMEM((_CAP,), jnp.float32),
            pltpu.VMEM((_K * 6 + 96,), jnp.float32),
            pltpu.VMEM((16,), jnp.int32),
        ],
    )(probs_pad, tbits, boxes_flat, scale_cols, classes_pad, labels_pad)


def kernel(pred_class_logits, pred_sim_logits, pred_boxes, orig_size, classes, labels):
    del pred_sim_logits
    B, N, C = pred_class_logits.shape
    probs_pad = _softmax_thresh(pred_class_logits)
    probs_pad = probs_pad.reshape(B, -1)
    tbits = _thresh(probs_pad).reshape(B, 128)

    boxes_flat = pred_boxes.reshape(B, N * 4)
    scale_cols = jnp.stack([orig_size[:, 0], orig_size[:, 1]], axis=0).astype(jnp.float32)
    classes_pad = jnp.pad(classes, ((0, 0), (0, 64 - classes.shape[1])))
    labels_pad = jnp.pad(labels, ((0, 0), (0, 64 - labels.shape[1])))

    out = _topk_sc(probs_pad, tbits, boxes_flat, scale_cols, classes_pad, labels_pad)
    return out.reshape(B, _K, 6)


# speedup vs baseline: 1.1501x; 1.0065x over previous
import functools

import jax
import jax.numpy as jnp
from jax import lax
from jax.experimental import pallas as pl
from jax.experimental.pallas import tpu as pltpu
from jax.experimental.pallas import tpu_sc as plsc

NUM_SELECT = 1000
_NB = 20000
_NBP = 20480


def _softmax_thresh_body(logits_ref, probs_ref, xt_ref):
    C = logits_ref.shape[2]
    x = logits_ref[0]
    xt_ref[...] = jnp.swapaxes(x, 0, 1)

    m = xt_ref[0]
    for j in range(1, C):
        m = jnp.maximum(m, xt_ref[j])
    s = None
    for st in range(0, C, 13):
        g = jnp.exp(xt_ref[st] - m)
        for j in range(st + 1, min(st + 13, C)):
            g = g + jnp.exp(xt_ref[j] - m)
        s = g if s is None else s + g
    prob = jnp.exp(xt_ref[C - 1] - m) / s
    padded = jnp.pad(prob, (0, _NBP - _NB))
    probs_ref[...] = padded.reshape(1, 1, 1, _NBP)


def _softmax_thresh(logits):
    B, N, C = logits.shape
    nchunks = N // _NB
    return pl.pallas_call(
        _softmax_thresh_body,
        grid=(B, nchunks),
        in_specs=[pl.BlockSpec((1, _NB, C), lambda b, n: (b, n, 0))],
        out_specs=[
            pl.BlockSpec((1, 1, 1, _NBP), lambda b, n: (b, n, 0, 0)),
        ],
        out_shape=[
            jax.ShapeDtypeStruct((B, nchunks, 1, _NBP), jnp.float32),
        ],
        scratch_shapes=[
            pltpu.VMEM((C, _NB), jnp.float32),
        ],
        compiler_params=pltpu.CompilerParams(
            dimension_semantics=("parallel", "parallel")),
    )(logits)[0]


def _thresh_body(probs_ref, tbits_ref):
    u = pltpu.bitcast(probs_ref[...], jnp.int32)

    def step(i, t):
        cand = t | (1 << (30 - i))
        cnt = jnp.sum((u >= cand).astype(jnp.int32), axis=1, keepdims=True)
        return jnp.where(cnt >= NUM_SELECT, cand, t)

    t = jax.lax.fori_loop(0, 31, step, jnp.zeros((u.shape[0], 1), jnp.int32))
    tbits_ref[...] = jnp.broadcast_to(t[:, :, None], tbits_ref.shape)


def _thresh(probs_pad):
    B, NPAD = probs_pad.shape
    return pl.pallas_call(
        _thresh_body,
        out_shape=jax.ShapeDtypeStruct((B, 1, 128), jnp.int32),
    )(probs_pad)


_K = NUM_SELECT
_CAP = 1024
_BUF = 1056


def _sc_body(probs_hbm, tbits_hbm, boxes_hbm, scale_hbm, cls_hbm, lbl_hbm,
             out_hbm,
             prob_v, tb_v, scl_v, cls_v, lbl_v, box_v,
             ka, ia, kc, ic, hist, base, gidx, score, outv, tmp16, sem):
    wid = lax.axis_index("s") * 2 + lax.axis_index("c")

    @pl.when(wid < 16)
    def _():
        t = wid
        pltpu.sync_copy(probs_hbm.at[t], prob_v)
        box_cp = pltpu.async_copy(boxes_hbm.at[t], box_v, sem)
        pltpu.sync_copy(tbits_hbm.at[t], tb_v)
        pltpu.sync_copy(cls_hbm.at[t], cls_v)
        pltpu.sync_copy(lbl_hbm.at[t], lbl_v)
        pltpu.sync_copy(scale_hbm, scl_v)

        lanes = lax.iota(jnp.int32, 16)
        tvec = tb_v[pl.ds(0, 16)]

        def compact(i, off):
            u = plsc.bitcast(prob_v[pl.ds(i * 16, 16)], jnp.int32)
            m = u >= tvec
            ks = ~u
            idx = i * 16 + lanes

            @pl.when(off < _CAP - 16)
            def _():
                plsc.store_compressed(ka.at[pl.ds(off, 16)], ks, mask=m)
                plsc.store_compressed(ia.at[pl.ds(off, 16)], idx, mask=m)

            return off + jnp.max(plsc.all_reduce_population_count(m))

        off = lax.fori_loop(0, prob_v.shape[0] // 16, compact, jnp.int32(0))

        def sentinel(j, _):
            @pl.when(off + j * 16 < _CAP + 16)
            def _():
                ka[pl.ds(off + j * 16, 16)] = jnp.full((16,), -1, jnp.int32)
                ia[pl.ds(off + j * 16, 16)] = jnp.zeros((16,), jnp.int32)
            return 0

        lax.fori_loop(0, 4, sentinel, 0)

        for p in range(6):
            src_k, src_i = (ka, ia) if p % 2 == 0 else (kc, ic)
            dst_k, dst_i = (kc, ic) if p % 2 == 0 else (ka, ia)
            sh = 5 * p
            hist[pl.ds(0, 16)] = jnp.zeros((16,), jnp.int32)
            hist[pl.ds(16, 16)] = jnp.zeros((16,), jnp.int32)

            def histo(v, _, src_k=src_k, sh=sh):
                k = plsc.bitcast(src_k[pl.ds(v * 16, 16)], jnp.uint32)
                d = plsc.bitcast((k >> sh) & 31, jnp.int32)
                cnt, lastm = plsc.scan_count(d)
                plsc.addupdate_scatter(hist, [d], cnt, mask=lastm)
                return 0

            lax.fori_loop(0, _CAP // 16, histo, 0)
            h0 = hist[pl.ds(0, 16)]
            h1 = hist[pl.ds(16, 16)]
            e0 = plsc.cumsum(h0) - h0
            tot0 = jnp.max(plsc.cumsum(h0))
            e1 = plsc.cumsum(h1) - h1 + tot0
            base[pl.ds(0, 16)] = e0
            base[pl.ds(16, 16)] = e1

            def permute(v, _, src_k=src_k, src_i=src_i, dst_k=dst_k,
                        dst_i=dst_i, sh=sh):
                k = src_k[pl.ds(v * 16, 16)]
                i_ = src_i[pl.ds(v * 16, 16)]
                d = plsc.bitcast((plsc.bitcast(k, jnp.uint32) >> sh) & 31,
                                 jnp.int32)
                cnt, lastm = plsc.scan_count(d)
                pos = plsc.load_gather(base, [d]) + (cnt - 1)
                plsc.store_scatter(dst_k, [pos], k)
                plsc.store_scatter(dst_i, [pos], i_)
                plsc.addupdate_scatter(base, [d], cnt, mask=lastm)
                return 0

            lax.fori_loop(0, _CAP // 16, permute, 0)
        res_k, res_i = ka, ia

        def post(v, _):
            ks = res_k[pl.ds(v * 16, 16)]
            prob = plsc.bitcast(~ks, jnp.float32)
            score[pl.ds(v * 16, 16)] = prob
            pidx = res_i[pl.ds(v * 16, 16)]
            orig = pidx - jnp.where(pidx >= _NBP,
                                    jnp.full((16,), _NBP - _NB, jnp.int32),
                                    jnp.zeros((16,), jnp.int32))
            gidx[pl.ds(v * 16, 16)] = orig
            return 0

        lax.fori_loop(0, _CAP // 16, post, 0)

        cid = jnp.zeros((16,), jnp.int32)
        for j in range(3, -1, -1):
            cls = cls_v[pl.ds(j * 16, 16)]
            lbl = lbl_v[pl.ds(j * 16, 16)]
            ids = jnp.where(lbl == 1, cls, jnp.zeros((16,), jnp.int32))
            m = ids != 0
            ffs = plsc.all_reduce_ffs(m)
            tmp16[pl.ds(0, 16)] = ids
            val = plsc.load_gather(tmp16, [jnp.minimum(ffs, 15)])
            cid = jnp.where(ffs < 16, val, cid)
        cid_f = cid.astype(jnp.float32)

        box_cp.wait()

        sh_s = plsc.load_gather(scl_v, [jnp.zeros((16,), jnp.int32),
                                        jnp.full((16,), t, jnp.int32)])
        sw_s = plsc.load_gather(scl_v, [jnp.ones((16,), jnp.int32),
                                        jnp.full((16,), t, jnp.int32)])

        def assemble(v, _):
            flat = v * 16 + lanes
            q = flat // 6
            c = flat - q * 6
            sc_g = plsc.load_gather(score, [q])
            ca = (c - 2) & 1
            cb = ca + 2
            oq = plsc.load_gather(gidx, [q])
            g1 = plsc.load_gather(box_v, [oq * 4 + ca])
            g2 = plsc.load_gather(box_v, [oq * 4 + cb])
            sgn = jnp.where(c < 4, jnp.full((16,), -0.5, jnp.float32),
                            jnp.full((16,), 0.5, jnp.float32))
            scl_l = jnp.where(ca == 0, sw_s, sh_s)
            bx = (g1 + sgn * g2) * scl_l
            val = jnp.where(c == 0, sc_g, jnp.where(c == 1, cid_f, bx))
            outv[pl.ds(v * 16, 16)] = val
            return 0

        lax.fori_loop(0, (_K * 6) // 16, assemble, 0)
        pltpu.sync_copy(outv.at[pl.ds(0, _K * 6)],
                        out_hbm.at[pl.ds(t * (_K * 6), _K * 6)])


def _topk_sc(probs_pad, tbits, boxes_flat, scale_cols, classes_pad, labels_pad):
    B = probs_pad.shape[0]
    mesh = plsc.VectorSubcoreMesh(core_axis_name="c", subcore_axis_name="s")
    return pl.kernel(
        _sc_body,
        out_type=jax.ShapeDtypeStruct((B * _K * 6,), jnp.float32),
        mesh=mesh,
        compiler_params=pltpu.CompilerParams(needs_layout_passes=False),
        scratch_types=[
            pltpu.VMEM((probs_pad.shape[1],), jnp.float32),
            pltpu.VMEM((128,), jnp.int32),
            pltpu.VMEM((2, 16), jnp.float32),
            pltpu.VMEM((64,), jnp.int32),
            pltpu.VMEM((64,), jnp.int32),
            pltpu.VMEM((80000,), jnp.float32),
            pltpu.VMEM((_BUF,), jnp.int32),
            pltpu.VMEM((_BUF,), jnp.int32),
            pltpu.VMEM((_CAP,), jnp.int32),
            pltpu.VMEM((_CAP,), jnp.int32),
            pltpu.VMEM((32,), jnp.int32),
            pltpu.VMEM((32,), jnp.int32),
            pltpu.VMEM((_CAP,), jnp.int32),
            pltpu.VMEM((_CAP,), jnp.float32),
            pltpu.VMEM((_K * 6 + 96,), jnp.float32),
            pltpu.VMEM((16,), jnp.int32),
            pltpu.SemaphoreType.DMA,
        ],
    )(probs_pad, tbits, boxes_flat, scale_cols, classes_pad, labels_pad)


def kernel(pred_class_logits, pred_sim_logits, pred_boxes, orig_size, classes, labels):
    del pred_sim_logits
    B, N, C = pred_class_logits.shape
    probs_pad = _softmax_thresh(pred_class_logits)
    probs_pad = probs_pad.reshape(B, -1)
    tbits = _thresh(probs_pad).reshape(B, 128)

    boxes_flat = pred_boxes.reshape(B, N * 4)
    scale_cols = jnp.stack([orig_size[:, 0], orig_size[:, 1]], axis=0).astype(jnp.float32)
    classes_pad = jnp.pad(classes, ((0, 0), (0, 64 - classes.shape[1])))
    labels_pad = jnp.pad(labels, ((0, 0), (0, 64 - labels.shape[1])))

    out = _topk_sc(probs_pad, tbits, boxes_flat, scale_cols, classes_pad, labels_pad)
    return out.reshape(B, _K, 6)
